# baseline (speedup 1.0000x reference)
import jax
import jax.numpy as jnp
from jax import lax
from jax.experimental import pallas as pl
from jax.experimental.pallas import tpu as pltpu
from jax.experimental.pallas import tpu_sc as plsc

H = 3
HID = 16
N = 400
F = 144
E = 3200
TB = 96
BBLK = 8
NCOL = TB * H
NW = 32
CPT = NCOL // NW
NCH = 3
CPC = CPT // NCH
EG = E // 16


def _cfull(v):
    return jnp.full((16,), v, dtype=jnp.int32)


def _sc_gat_body(src_h, dst_h, el_h, er_h, m_h, h_h, out_h,
                 src_v, dst_v, el_v, er_v, esum_v, m_v, h_v, out_v):
    wid = lax.axis_index("s") * 2 + lax.axis_index("c")
    pltpu.sync_copy(src_h, src_v)
    pltpu.sync_copy(dst_h, dst_v)
    pltpu.sync_copy(el_h.at[wid], el_v)
    pltpu.sync_copy(er_h.at[wid], er_v)
    pltpu.sync_copy(m_h.at[wid], m_v)
    zero16 = jnp.zeros((16,), jnp.float32)

    def zsum(i, _):
        off = pl.multiple_of(i * 16, 16)
        for j in range(CPT):
            esum_v[j, pl.ds(off, 16)] = zero16
        return 0

    lax.fori_loop(0, N // 16, zsum, 0)

    mj = [m_v[j] for j in range(CPT)]

    def pass1(g, _):
        off = pl.multiple_of(g * 16, 16)
        sv = src_v[pl.ds(off, 16)]
        dv = dst_v[pl.ds(off, 16)]
        for j in range(CPT):
            ele = plsc.load_gather(el_v, [_cfull(j), sv])
            ere = plsc.load_gather(er_v, [_cfull(j), dv])
            ev = ele + ere
            ev = jnp.where(ev >= 0, ev, 0.2 * ev)
            plsc.addupdate_scatter(esum_v, [_cfull(j), dv], jnp.exp(ev - mj[j]))
        return 0

    lax.fori_loop(0, EG, pass1, 0)

    for cc in range(NCH):
        pltpu.sync_copy(h_h.at[wid, cc], h_v)

        def zout(i, _):
            off = pl.multiple_of(i * 16, 16)
            for j in range(CPC):
                for k in range(HID):
                    out_v[j, k, pl.ds(off, 16)] = zero16
            return 0

        lax.fori_loop(0, N // 16, zout, 0)

        def pass2(g, _, cc=cc):
            off = pl.multiple_of(g * 16, 16)
            sv = src_v[pl.ds(off, 16)]
            dv = dst_v[pl.ds(off, 16)]
            for j in range(CPC):
                jj = _cfull(cc * CPC + j)
                ele = plsc.load_gather(el_v, [jj, sv])
                ere = plsc.load_gather(er_v, [jj, dv])
                ev = ele + ere
                ev = jnp.where(ev >= 0, ev, 0.2 * ev)
                ee = jnp.exp(ev - mj[cc * CPC + j])
                es = plsc.load_gather(esum_v, [jj, dv])
                alpha = ee / es
                for k in range(HID):
                    hv = plsc.load_gather(h_v, [_cfull(j), _cfull(k), sv])
                    plsc.addupdate_scatter(out_v, [_cfull(j), _cfull(k), dv],
                                           alpha * hv)
            return 0

        lax.fori_loop(0, EG, pass2, 0)
        pltpu.sync_copy(out_v, out_h.at[wid, cc])


def _sc_gat_edge(src, dst, el_g, er_g, m_g, h_g):
    return pl.kernel(
        _sc_gat_body,
        out_type=jax.ShapeDtypeStruct((NW, NCH, CPC, HID, N), jnp.float32),
        mesh=plsc.VectorSubcoreMesh(core_axis_name="c", subcore_axis_name="s"),
        compiler_params=pltpu.CompilerParams(needs_layout_passes=False),
        scratch_types=[
            pltpu.VMEM((E,), jnp.int32),
            pltpu.VMEM((E,), jnp.int32),
            pltpu.VMEM((CPT, N), jnp.float32),
            pltpu.VMEM((CPT, N), jnp.float32),
            pltpu.VMEM((CPT, N), jnp.float32),
            pltpu.VMEM((CPT, 16), jnp.float32),
            pltpu.VMEM((CPC, HID, N), jnp.float32),
            pltpu.VMEM((CPC, HID, N), jnp.float32),
        ],
    )(src, dst, el_g, er_g, m_g, h_g)


def _dense_h_kernel(in_ref, cov_ref, wd_ref, wm_ref, ws_ref,
                    ald_ref, alm_ref, als_ref, ard_ref, arm_ref, ars_ref,
                    hd_ref, hm_ref, hs_ref,
                    eld_ref, elm_ref, els_ref, erd_ref, erm_ref, ers_ref):
    x = in_ref[...] + cov_ref[...]
    for w_ref, al_ref, ar_ref, h_ref, el_ref, er_ref in (
            (wd_ref, ald_ref, ard_ref, hd_ref, eld_ref, erd_ref),
            (wm_ref, alm_ref, arm_ref, hm_ref, elm_ref, erm_ref),
            (ws_ref, als_ref, ars_ref, hs_ref, els_ref, ers_ref)):
        h = jnp.dot(x, w_ref[...], preferred_element_type=jnp.float32)
        h_ref[...] = h
        el_ref[...] = jnp.dot(h, al_ref[...], preferred_element_type=jnp.float32)
        er_ref[...] = jnp.dot(h, ar_ref[...], preferred_element_type=jnp.float32)


def _attn_kernel(cov_ref, out_ref, acc_ref):
    t = pl.program_id(1)

    @pl.when(t == 0)
    def _():
        c = cov_ref[0]
        a = lax.dot_general(c, c, (((1,), (1,)), ((), ())),
                            preferred_element_type=jnp.float32)
        m = jnp.max(a, axis=1, keepdims=True)
        p = jnp.exp(a - m)
        acc_ref[...] = p / jnp.sum(p, axis=1, keepdims=True)

    out_ref[0] = acc_ref[...]


def _expand_al(al):
    flat = al.reshape(48)
    cols = jnp.arange(48) // HID
    onehot = (jnp.arange(8)[None, :] == cols[:, None]).astype(jnp.float32)
    return flat[:, None] * onehot


def kernel(input, covariate, edge_index_d, W_d, al_d, ar_d, b_d,
           edge_index_m, W_m, al_m, ar_m, b_m,
           edge_index_s, W_s, al_s, ar_s, b_s):
    batch, T = input.shape[0], input.shape[1]
    in_r = input.reshape(TB * N, F)
    cov_r = covariate.reshape(TB * N, F)

    blk = N * BBLK
    grid_a = (TB * N) // blk
    als = [_expand_al(a) for a in (al_d, al_m, al_s)]
    ars = [_expand_al(a) for a in (ar_d, ar_m, ar_s)]
    row_spec = pl.BlockSpec((blk, F), lambda i: (i, 0))
    w_spec = pl.BlockSpec((F, 48), lambda i: (0, 0))
    a_spec = pl.BlockSpec((48, 8), lambda i: (0, 0))
    h_spec = pl.BlockSpec((blk, 48), lambda i: (i, 0))
    e_spec = pl.BlockSpec((blk, 8), lambda i: (i, 0))
    res = pl.pallas_call(
        _dense_h_kernel,
        grid=(grid_a,),
        out_shape=[jax.ShapeDtypeStruct((TB * N, 48), jnp.float32)] * 3
        + [jax.ShapeDtypeStruct((TB * N, 8), jnp.float32)] * 6,
        in_specs=[row_spec, row_spec, w_spec, w_spec, w_spec,
                  a_spec, a_spec, a_spec, a_spec, a_spec, a_spec],
        out_specs=[h_spec] * 3 + [e_spec] * 6,
    )(in_r, cov_r, W_d, W_m, W_s, *als, *ars)
    h_gs = res[:3]
    el_gs = res[3:6]
    er_gs = res[6:9]

    outs = []
    for g, (edge_index, b) in enumerate([(edge_index_d, b_d),
                                         (edge_index_m, b_m),
                                         (edge_index_s, b_s)]):
        src = edge_index[0].astype(jnp.int32)
        dst = edge_index[1].astype(jnp.int32)
        el3 = el_gs[g].reshape(TB, N, 8)[:, :, :H]
        er3 = er_gs[g].reshape(TB, N, 8)[:, :, :H]
        el_g = el3.transpose(0, 2, 1).reshape(NW, CPT, N)
        er_g = er3.transpose(0, 2, 1).reshape(NW, CPT, N)
        ms = jnp.max(el3, axis=1) + jnp.max(er3, axis=1)
        m_g = jnp.broadcast_to(
            jnp.where(ms >= 0, ms, 0.2 * ms).reshape(NW, CPT, 1), (NW, CPT, 16))
        h_g = (h_gs[g].reshape(TB, N, H, HID).transpose(0, 2, 3, 1)
               .reshape(NW, NCH, CPC, HID, N))
        out_sc = _sc_gat_edge(src, dst, el_g, er_g, m_g, h_g)
        out_g = (out_sc.reshape(TB, H, HID, N).transpose(3, 0, 1, 2))
        outs.append(out_g + b.reshape(1, 1, H, HID))

    x_attn = jnp.concatenate(outs, axis=-1)
    x_attn = jnp.transpose(x_attn, (1, 0, 2, 3)).reshape(batch, T, N, F)
    x_attn = jnp.transpose(x_attn, (1, 0, 2, 3)).reshape(batch, T, N, F)
    out = input + x_attn

    cov0 = covariate[:, 0]
    attn = pl.pallas_call(
        _attn_kernel,
        grid=(batch, T),
        out_shape=jax.ShapeDtypeStruct((batch * T, N, N), jnp.float32),
        in_specs=[pl.BlockSpec((1, N, F), lambda bb, t: (bb, 0, 0))],
        out_specs=pl.BlockSpec((1, N, N), lambda bb, t: (t * batch + bb, 0, 0)),
        scratch_shapes=[pltpu.VMEM((N, N), jnp.float32)],
    )(cov0)
    return out, attn


# speedup vs baseline: 1.0156x; 1.0156x over previous
import jax
import jax.numpy as jnp
from jax import lax
from jax.experimental import pallas as pl
from jax.experimental.pallas import tpu as pltpu
from jax.experimental.pallas import tpu_sc as plsc

H = 3
HID = 16
N = 400
F = 144
E = 3200
TB = 96
BBLK = 8
NCOL = TB * H
NW = 32
CPT = NCOL // NW
NCH = 3
CPC = CPT // NCH
EG = E // 16


def _cfull(v):
    return jnp.full((16,), v, dtype=jnp.int32)


def _sc_gat_body(src_h, dst_h, el_h, er_h, m_h, h_h, out_h,
                 src_v, dst_v, el_v, er_v, esum_v, m_v, h_v, out_v):
    wid = lax.axis_index("s") * 2 + lax.axis_index("c")
    pltpu.sync_copy(src_h, src_v)
    pltpu.sync_copy(dst_h, dst_v)
    pltpu.sync_copy(el_h.at[wid], el_v)
    pltpu.sync_copy(er_h.at[wid], er_v)
    pltpu.sync_copy(m_h.at[wid], m_v)
    zero16 = jnp.zeros((16,), jnp.float32)

    def zsum(i, _):
        off = pl.multiple_of(i * 16, 16)
        for j in range(CPT):
            esum_v[j, pl.ds(off, 16)] = zero16
        return 0

    lax.fori_loop(0, N // 16, zsum, 0)

    mj = [m_v[j] for j in range(CPT)]

    def pass1(g, _):
        off = pl.multiple_of(g * 16, 16)
        sv = src_v[pl.ds(off, 16)]
        dv = dst_v[pl.ds(off, 16)]
        for j in range(CPT):
            ele = plsc.load_gather(el_v, [_cfull(j), sv])
            ere = plsc.load_gather(er_v, [_cfull(j), dv])
            ev = ele + ere
            ev = jnp.where(ev >= 0, ev, 0.2 * ev)
            plsc.addupdate_scatter(esum_v, [_cfull(j), dv], jnp.exp(ev - mj[j]))
        return 0

    lax.fori_loop(0, EG, pass1, 0)

    for cc in range(NCH):
        pltpu.sync_copy(h_h.at[wid, cc], h_v)

        def zout(i, _):
            off = pl.multiple_of(i * 16, 16)
            for j in range(CPC):
                for k in range(HID):
                    out_v[j, k, pl.ds(off, 16)] = zero16
            return 0

        lax.fori_loop(0, N // 16, zout, 0)

        def pass2(g, _, cc=cc):
            off = pl.multiple_of(g * 16, 16)
            sv = src_v[pl.ds(off, 16)]
            dv = dst_v[pl.ds(off, 16)]
            for j in range(CPC):
                jj = _cfull(cc * CPC + j)
                ele = plsc.load_gather(el_v, [jj, sv])
                ere = plsc.load_gather(er_v, [jj, dv])
                ev = ele + ere
                ev = jnp.where(ev >= 0, ev, 0.2 * ev)
                ee = jnp.exp(ev - mj[cc * CPC + j])
                es = plsc.load_gather(esum_v, [jj, dv])
                alpha = ee / es
                for k in range(HID):
                    hv = plsc.load_gather(h_v, [_cfull(j), _cfull(k), sv])
                    plsc.addupdate_scatter(out_v, [_cfull(j), _cfull(k), dv],
                                           alpha * hv)
            return 0

        lax.fori_loop(0, EG, pass2, 0)
        pltpu.sync_copy(out_v, out_h.at[wid, cc])


def _sc_gat_edge(src, dst, el_g, er_g, m_g, h_g):
    return pl.kernel(
        _sc_gat_body,
        out_type=jax.ShapeDtypeStruct((NW, NCH, CPC, HID, N), jnp.float32),
        mesh=plsc.VectorSubcoreMesh(core_axis_name="c", subcore_axis_name="s"),
        compiler_params=pltpu.CompilerParams(needs_layout_passes=False),
        scratch_types=[
            pltpu.VMEM((E,), jnp.int32),
            pltpu.VMEM((E,), jnp.int32),
            pltpu.VMEM((CPT, N), jnp.float32),
            pltpu.VMEM((CPT, N), jnp.float32),
            pltpu.VMEM((CPT, N), jnp.float32),
            pltpu.VMEM((CPT, 16), jnp.float32),
            pltpu.VMEM((CPC, HID, N), jnp.float32),
            pltpu.VMEM((CPC, HID, N), jnp.float32),
        ],
    )(src, dst, el_g, er_g, m_g, h_g)


def _dense_h_kernel(in_ref, cov_ref, wd_ref, wm_ref, ws_ref,
                    ald_ref, alm_ref, als_ref, ard_ref, arm_ref, ars_ref,
                    hd_ref, hm_ref, hs_ref,
                    eld_ref, elm_ref, els_ref, erd_ref, erm_ref, ers_ref):
    x = in_ref[...] + cov_ref[...]
    for w_ref, al_ref, ar_ref, h_ref, el_ref, er_ref in (
            (wd_ref, ald_ref, ard_ref, hd_ref, eld_ref, erd_ref),
            (wm_ref, alm_ref, arm_ref, hm_ref, elm_ref, erm_ref),
            (ws_ref, als_ref, ars_ref, hs_ref, els_ref, ers_ref)):
        h = jnp.dot(x, w_ref[...], preferred_element_type=jnp.float32)
        el = jnp.dot(h, al_ref[...], preferred_element_type=jnp.float32)
        er = jnp.dot(h, ar_ref[...], preferred_element_type=jnp.float32)
        h_ref[...] = (h.reshape(BBLK, N, 48).transpose(0, 2, 1)
                      .reshape(BBLK * 48, N))
        el_ref[...] = (el.reshape(BBLK, N, 8).transpose(0, 2, 1)
                       .reshape(BBLK * 8, N))
        er_ref[...] = (er.reshape(BBLK, N, 8).transpose(0, 2, 1)
                       .reshape(BBLK * 8, N))


def _out_t_kernel(od_ref, om_ref, os_ref, td_ref, tm_ref, ts_ref):
    for o_ref, t_ref in ((od_ref, td_ref), (om_ref, tm_ref), (os_ref, ts_ref)):
        t_ref[...] = (o_ref[...].reshape(BBLK, 48, N).transpose(0, 2, 1)
                      .reshape(BBLK * N, 48))


def _attn_kernel(cov_ref, out_ref, acc_ref):
    t = pl.program_id(1)

    @pl.when(t == 0)
    def _():
        c = cov_ref[0]
        a = lax.dot_general(c, c, (((1,), (1,)), ((), ())),
                            preferred_element_type=jnp.float32)
        m = jnp.max(a, axis=1, keepdims=True)
        p = jnp.exp(a - m)
        acc_ref[...] = p / jnp.sum(p, axis=1, keepdims=True)

    out_ref[0] = acc_ref[...]


def _expand_al(al):
    flat = al.reshape(48)
    cols = jnp.arange(48) // HID
    onehot = (jnp.arange(8)[None, :] == cols[:, None]).astype(jnp.float32)
    return flat[:, None] * onehot


def kernel(input, covariate, edge_index_d, W_d, al_d, ar_d, b_d,
           edge_index_m, W_m, al_m, ar_m, b_m,
           edge_index_s, W_s, al_s, ar_s, b_s):
    batch, T = input.shape[0], input.shape[1]
    in_r = input.reshape(TB * N, F)
    cov_r = covariate.reshape(TB * N, F)

    blk = N * BBLK
    grid_a = (TB * N) // blk
    als = [_expand_al(a) for a in (al_d, al_m, al_s)]
    ars = [_expand_al(a) for a in (ar_d, ar_m, ar_s)]
    row_spec = pl.BlockSpec((blk, F), lambda i: (i, 0))
    w_spec = pl.BlockSpec((F, 48), lambda i: (0, 0))
    a_spec = pl.BlockSpec((48, 8), lambda i: (0, 0))
    h_spec = pl.BlockSpec((BBLK * 48, N), lambda i: (i, 0))
    e_spec = pl.BlockSpec((BBLK * 8, N), lambda i: (i, 0))
    res = pl.pallas_call(
        _dense_h_kernel,
        grid=(grid_a,),
        out_shape=[jax.ShapeDtypeStruct((TB * 48, N), jnp.float32)] * 3
        + [jax.ShapeDtypeStruct((TB * 8, N), jnp.float32)] * 6,
        in_specs=[row_spec, row_spec, w_spec, w_spec, w_spec,
                  a_spec, a_spec, a_spec, a_spec, a_spec, a_spec],
        out_specs=[h_spec] * 3 + [e_spec] * 6,
    )(in_r, cov_r, W_d, W_m, W_s, *als, *ars)
    h_gs = res[:3]
    el_gs = res[3:6]
    er_gs = res[6:9]

    out_scs = []
    for g, edge_index in enumerate([edge_index_d, edge_index_m, edge_index_s]):
        src = edge_index[0].astype(jnp.int32)
        dst = edge_index[1].astype(jnp.int32)
        el3 = el_gs[g].reshape(TB, 8, N)[:, :H]
        er3 = er_gs[g].reshape(TB, 8, N)[:, :H]
        el_g = el3.reshape(NW, CPT, N)
        er_g = er3.reshape(NW, CPT, N)
        ms = jnp.max(el3, axis=2) + jnp.max(er3, axis=2)
        m_g = jnp.broadcast_to(
            jnp.where(ms >= 0, ms, 0.2 * ms).reshape(NW, CPT, 1), (NW, CPT, 16))
        h_g = h_gs[g].reshape(NW, NCH, CPC, HID, N)
        out_scs.append(_sc_gat_edge(src, dst, el_g, er_g, m_g, h_g)
                       .reshape(TB * 48, N))

    o_spec = pl.BlockSpec((BBLK * 48, N), lambda i: (i, 0))
    t_spec = pl.BlockSpec((blk, 48), lambda i: (i, 0))
    outs_t = pl.pallas_call(
        _out_t_kernel,
        grid=(grid_a,),
        out_shape=[jax.ShapeDtypeStruct((TB * N, 48), jnp.float32)] * 3,
        in_specs=[o_spec] * 3,
        out_specs=[t_spec] * 3,
    )(*out_scs)
    outs = [outs_t[g].reshape(TB, N, H, HID) + b.reshape(1, 1, H, HID)
            for g, b in enumerate((b_d, b_m, b_s))]

    x_attn = jnp.concatenate(outs, axis=-1)
    x_attn = x_attn.reshape(batch, T, N, F)
    x_attn = jnp.transpose(x_attn, (1, 0, 2, 3)).reshape(batch, T, N, F)
    out = input + x_attn

    cov0 = covariate[:, 0]
    attn = pl.pallas_call(
        _attn_kernel,
        grid=(batch, T),
        out_shape=jax.ShapeDtypeStruct((batch * T, N, N), jnp.float32),
        in_specs=[pl.BlockSpec((1, N, F), lambda bb, t: (bb, 0, 0))],
        out_specs=pl.BlockSpec((1, N, N), lambda bb, t: (t * batch + bb, 0, 0)),
        scratch_shapes=[pltpu.VMEM((N, N), jnp.float32)],
    )(cov0)
    return out, attn


# speedup vs baseline: 1.0396x; 1.0237x over previous
import jax
import jax.numpy as jnp
from jax import lax
from jax.experimental import pallas as pl
from jax.experimental.pallas import tpu as pltpu
from jax.experimental.pallas import tpu_sc as plsc

H = 3
HID = 16
N = 400
F = 144
E = 3200
TB = 96
BBLK = 8
NCOL = TB * H
NW = 32
CPT = NCOL // NW
NCH = 3
CPC = CPT // NCH
EG = E // 16


def _cfull(v):
    return jnp.full((16,), v, dtype=jnp.int32)


def _sc_gat_body(src_h, dst_h, el_h, er_h, m_h, h_h, out_h,
                 src_v, dst_v, el_v, er_v, esum_v, m_v, h_v, out_v):
    wid = lax.axis_index("s") * 2 + lax.axis_index("c")
    pltpu.sync_copy(src_h, src_v)
    pltpu.sync_copy(dst_h, dst_v)
    pltpu.sync_copy(el_h.at[wid], el_v)
    pltpu.sync_copy(er_h.at[wid], er_v)
    pltpu.sync_copy(m_h.at[wid], m_v)
    zero16 = jnp.zeros((16,), jnp.float32)

    def zsum(i, _):
        off = pl.multiple_of(i * 16, 16)
        for j in range(CPT):
            esum_v[j, pl.ds(off, 16)] = zero16
        return 0

    lax.fori_loop(0, N // 16, zsum, 0)

    mj = [m_v[j] for j in range(CPT)]

    def pass1(g, _):
        off = pl.multiple_of(g * 16, 16)
        sv = src_v[pl.ds(off, 16)]
        dv = dst_v[pl.ds(off, 16)]
        for j in range(CPT):
            ele = plsc.load_gather(el_v, [_cfull(j), sv])
            ere = plsc.load_gather(er_v, [_cfull(j), dv])
            ev = ele + ere
            ev = jnp.where(ev >= 0, ev, 0.2 * ev)
            plsc.addupdate_scatter(esum_v, [_cfull(j), dv], jnp.exp(ev - mj[j]))
        return 0

    lax.fori_loop(0, EG, pass1, 0)

    for cc in range(NCH):
        pltpu.sync_copy(h_h.at[wid, cc], h_v)

        def zout(i, _):
            off = pl.multiple_of(i * 16, 16)
            for j in range(CPC):
                for k in range(HID):
                    out_v[j, k, pl.ds(off, 16)] = zero16
            return 0

        lax.fori_loop(0, N // 16, zout, 0)

        def pass2(g, _, cc=cc):
            off = pl.multiple_of(g * 16, 16)
            sv = src_v[pl.ds(off, 16)]
            dv = dst_v[pl.ds(off, 16)]
            for j in range(CPC):
                jj = _cfull(cc * CPC + j)
                ele = plsc.load_gather(el_v, [jj, sv])
                ere = plsc.load_gather(er_v, [jj, dv])
                ev = ele + ere
                ev = jnp.where(ev >= 0, ev, 0.2 * ev)
                ee = jnp.exp(ev - mj[cc * CPC + j])
                es = plsc.load_gather(esum_v, [jj, dv])
                alpha = ee / es
                for k in range(HID):
                    hv = plsc.load_gather(h_v, [_cfull(j), _cfull(k), sv])
                    plsc.addupdate_scatter(out_v, [_cfull(j), _cfull(k), dv],
                                           alpha * hv)
            return 0

        lax.fori_loop(0, EG, pass2, 0)
        pltpu.sync_copy(out_v, out_h.at[wid, cc])


def _sc_gat_edge(src, dst, el_g, er_g, m_g, h_g):
    return pl.kernel(
        _sc_gat_body,
        out_type=jax.ShapeDtypeStruct((NW, NCH, CPC, HID, N), jnp.float32),
        mesh=plsc.VectorSubcoreMesh(core_axis_name="c", subcore_axis_name="s"),
        compiler_params=pltpu.CompilerParams(needs_layout_passes=False),
        scratch_types=[
            pltpu.VMEM((E,), jnp.int32),
            pltpu.VMEM((E,), jnp.int32),
            pltpu.VMEM((CPT, N), jnp.float32),
            pltpu.VMEM((CPT, N), jnp.float32),
            pltpu.VMEM((CPT, N), jnp.float32),
            pltpu.VMEM((CPT, 16), jnp.float32),
            pltpu.VMEM((CPC, HID, N), jnp.float32),
            pltpu.VMEM((CPC, HID, N), jnp.float32),
        ],
    )(src, dst, el_g, er_g, m_g, h_g)


def _dense_h_kernel(in_ref, cov_ref, wd_ref, wm_ref, ws_ref,
                    ald_ref, alm_ref, als_ref, ard_ref, arm_ref, ars_ref,
                    hd_ref, hm_ref, hs_ref,
                    eld_ref, elm_ref, els_ref, erd_ref, erm_ref, ers_ref):
    x = in_ref[...] + cov_ref[...]
    for w_ref, al_ref, ar_ref, h_ref, el_ref, er_ref in (
            (wd_ref, ald_ref, ard_ref, hd_ref, eld_ref, erd_ref),
            (wm_ref, alm_ref, arm_ref, hm_ref, elm_ref, erm_ref),
            (ws_ref, als_ref, ars_ref, hs_ref, els_ref, ers_ref)):
        h = jnp.dot(x, w_ref[...], preferred_element_type=jnp.float32)
        el = jnp.dot(h, al_ref[...], preferred_element_type=jnp.float32)
        er = jnp.dot(h, ar_ref[...], preferred_element_type=jnp.float32)
        h_ref[...] = (h.reshape(BBLK, N, 48).transpose(0, 2, 1)
                      .reshape(BBLK * 48, N))
        el_ref[...] = (el.reshape(BBLK, N, 8).transpose(0, 2, 1)
                       .reshape(BBLK * 8, N))
        er_ref[...] = (er.reshape(BBLK, N, 8).transpose(0, 2, 1)
                       .reshape(BBLK * 8, N))


def _assemble_kernel(od_ref, om_ref, os_ref, resid_ref, bias_ref, out_ref):
    parts = []
    for head in range(H):
        for o_ref in (od_ref, om_ref, os_ref):
            parts.append(o_ref[pl.ds(head * HID, HID), :].transpose(1, 0))
    x = jnp.concatenate(parts, axis=1)
    out_ref[...] = resid_ref[...] + x + bias_ref[...]


def _attn_kernel(cov_ref, out_ref, acc_ref):
    t = pl.program_id(1)

    @pl.when(t == 0)
    def _():
        c = cov_ref[0]
        a = lax.dot_general(c, c, (((1,), (1,)), ((), ())),
                            preferred_element_type=jnp.float32)
        m = jnp.max(a, axis=1, keepdims=True)
        p = jnp.exp(a - m)
        acc_ref[...] = p / jnp.sum(p, axis=1, keepdims=True)

    out_ref[0] = acc_ref[...]


def _expand_al(al):
    flat = al.reshape(48)
    cols = jnp.arange(48) // HID
    onehot = (jnp.arange(8)[None, :] == cols[:, None]).astype(jnp.float32)
    return flat[:, None] * onehot


def kernel(input, covariate, edge_index_d, W_d, al_d, ar_d, b_d,
           edge_index_m, W_m, al_m, ar_m, b_m,
           edge_index_s, W_s, al_s, ar_s, b_s):
    batch, T = input.shape[0], input.shape[1]
    in_r = input.reshape(TB * N, F)
    cov_r = covariate.reshape(TB * N, F)

    blk = N * BBLK
    grid_a = (TB * N) // blk
    als = [_expand_al(a) for a in (al_d, al_m, al_s)]
    ars = [_expand_al(a) for a in (ar_d, ar_m, ar_s)]
    row_spec = pl.BlockSpec((blk, F), lambda i: (i, 0))
    w_spec = pl.BlockSpec((F, 48), lambda i: (0, 0))
    a_spec = pl.BlockSpec((48, 8), lambda i: (0, 0))
    h_spec = pl.BlockSpec((BBLK * 48, N), lambda i: (i, 0))
    e_spec = pl.BlockSpec((BBLK * 8, N), lambda i: (i, 0))
    res = pl.pallas_call(
        _dense_h_kernel,
        grid=(grid_a,),
        out_shape=[jax.ShapeDtypeStruct((TB * 48, N), jnp.float32)] * 3
        + [jax.ShapeDtypeStruct((TB * 8, N), jnp.float32)] * 6,
        in_specs=[row_spec, row_spec, w_spec, w_spec, w_spec,
                  a_spec, a_spec, a_spec, a_spec, a_spec, a_spec],
        out_specs=[h_spec] * 3 + [e_spec] * 6,
    )(in_r, cov_r, W_d, W_m, W_s, *als, *ars)
    h_gs = res[:3]
    el_gs = res[3:6]
    er_gs = res[6:9]

    out_scs = []
    for g, edge_index in enumerate([edge_index_d, edge_index_m, edge_index_s]):
        src = edge_index[0].astype(jnp.int32)
        dst = edge_index[1].astype(jnp.int32)
        el3 = el_gs[g].reshape(TB, 8, N)[:, :H]
        er3 = er_gs[g].reshape(TB, 8, N)[:, :H]
        el_g = el3.reshape(NW, CPT, N)
        er_g = er3.reshape(NW, CPT, N)
        ms = jnp.max(el3, axis=2) + jnp.max(er3, axis=2)
        m_g = jnp.broadcast_to(
            jnp.where(ms >= 0, ms, 0.2 * ms).reshape(NW, CPT, 1), (NW, CPT, 16))
        h_g = h_gs[g].reshape(NW, NCH, CPC, HID, N)
        out_scs.append(_sc_gat_edge(src, dst, el_g, er_g, m_g, h_g)
                       .reshape(TB * 48, N))

    bias_cat = jnp.stack(
        [b.reshape(H, HID) for b in (b_d, b_m, b_s)], axis=1).reshape(1, F)

    def _q(bb, t):
        p = bb * T + t
        return (p % batch) * T + p // batch

    o_spec = pl.BlockSpec((48, N), lambda bb, t: (_q(bb, t), 0))
    out = pl.pallas_call(
        _assemble_kernel,
        grid=(batch, T),
        out_shape=jax.ShapeDtypeStruct((TB * N, F), jnp.float32),
        in_specs=[o_spec, o_spec, o_spec,
                  pl.BlockSpec((N, F), lambda bb, t: (bb * T + t, 0)),
                  pl.BlockSpec((1, F), lambda bb, t: (0, 0))],
        out_specs=pl.BlockSpec((N, F), lambda bb, t: (bb * T + t, 0)),
    )(*out_scs, in_r, bias_cat).reshape(batch, T, N, F)

    cov0 = covariate[:, 0]
    attn = pl.pallas_call(
        _attn_kernel,
        grid=(batch, T),
        out_shape=jax.ShapeDtypeStruct((batch * T, N, N), jnp.float32),
        in_specs=[pl.BlockSpec((1, N, F), lambda bb, t: (bb, 0, 0))],
        out_specs=pl.BlockSpec((1, N, N), lambda bb, t: (t * batch + bb, 0, 0)),
        scratch_shapes=[pltpu.VMEM((N, N), jnp.float32)],
    )(cov0)
    return out, attn


# speedup vs baseline: 1.0404x; 1.0007x over previous
import jax
import jax.numpy as jnp
from jax import lax
from jax.experimental import pallas as pl
from jax.experimental.pallas import tpu as pltpu
from jax.experimental.pallas import tpu_sc as plsc

H = 3
HID = 16
N = 400
F = 144
E = 3200
TB = 96
BBLK = 8
NCOL = TB * H
NW = 32
CPT = NCOL // NW
NCH = 3
CPC = CPT // NCH
EG = E // 16


def _cfull(v):
    return jnp.full((16,), v, dtype=jnp.int32)


def _sc_gat_body(src_h, dst_h, el_h, er_h, m_h, h_h, out_h,
                 src_v, dst_v, el_v, er_v, esum_v, m_v, h_v, out_v):
    wid = lax.axis_index("s") * 2 + lax.axis_index("c")
    pltpu.sync_copy(src_h, src_v)
    pltpu.sync_copy(dst_h, dst_v)
    pltpu.sync_copy(el_h.at[pl.ds(wid * (NCH * 8), NCH * 8)], el_v)
    pltpu.sync_copy(er_h.at[pl.ds(wid * (NCH * 8), NCH * 8)], er_v)
    pltpu.sync_copy(m_h.at[wid], m_v)
    zero16 = jnp.zeros((16,), jnp.float32)

    def _lrow(j):
        return _cfull((j // H) * 8 + j % H)

    def zsum(i, _):
        off = pl.multiple_of(i * 16, 16)
        for j in range(CPT):
            esum_v[j, pl.ds(off, 16)] = zero16
        return 0

    lax.fori_loop(0, N // 16, zsum, 0)

    mj = [m_v[j] for j in range(CPT)]

    def pass1(g, _):
        off = pl.multiple_of(g * 16, 16)
        sv = src_v[pl.ds(off, 16)]
        dv = dst_v[pl.ds(off, 16)]
        for j in range(CPT):
            ele = plsc.load_gather(el_v, [_lrow(j), sv])
            ere = plsc.load_gather(er_v, [_lrow(j), dv])
            ev = ele + ere
            ev = jnp.where(ev >= 0, ev, 0.2 * ev)
            plsc.addupdate_scatter(esum_v, [_cfull(j), dv], jnp.exp(ev - mj[j]))
        return 0

    lax.fori_loop(0, EG, pass1, 0)

    for cc in range(NCH):
        pltpu.sync_copy(h_h.at[pl.ds((wid * NCH + cc) * 48, 48)], h_v)

        def zout(i, _):
            off = pl.multiple_of(i * 16, 16)
            for r in range(48):
                out_v[r, pl.ds(off, 16)] = zero16
            return 0

        lax.fori_loop(0, N // 16, zout, 0)

        def pass2(g, _, cc=cc):
            off = pl.multiple_of(g * 16, 16)
            sv = src_v[pl.ds(off, 16)]
            dv = dst_v[pl.ds(off, 16)]
            for j in range(CPC):
                jj = _cfull(cc * CPC + j)
                ele = plsc.load_gather(el_v, [_lrow(cc * CPC + j), sv])
                ere = plsc.load_gather(er_v, [_lrow(cc * CPC + j), dv])
                ev = ele + ere
                ev = jnp.where(ev >= 0, ev, 0.2 * ev)
                ee = jnp.exp(ev - mj[cc * CPC + j])
                es = plsc.load_gather(esum_v, [jj, dv])
                alpha = ee / es
                for k in range(HID):
                    hv = plsc.load_gather(h_v, [_cfull(j * HID + k), sv])
                    plsc.addupdate_scatter(out_v, [_cfull(j * HID + k), dv],
                                           alpha * hv)
            return 0

        lax.fori_loop(0, EG, pass2, 0)
        pltpu.sync_copy(out_v, out_h.at[pl.ds((wid * NCH + cc) * 48, 48)])


def _sc_gat_edge(src, dst, el_g, er_g, m_g, h_g):
    return pl.kernel(
        _sc_gat_body,
        out_type=jax.ShapeDtypeStruct((TB * 48, N), jnp.float32),
        mesh=plsc.VectorSubcoreMesh(core_axis_name="c", subcore_axis_name="s"),
        compiler_params=pltpu.CompilerParams(needs_layout_passes=False),
        scratch_types=[
            pltpu.VMEM((E,), jnp.int32),
            pltpu.VMEM((E,), jnp.int32),
            pltpu.VMEM((NCH * 8, N), jnp.float32),
            pltpu.VMEM((NCH * 8, N), jnp.float32),
            pltpu.VMEM((CPT, N), jnp.float32),
            pltpu.VMEM((CPT, 16), jnp.float32),
            pltpu.VMEM((48, N), jnp.float32),
            pltpu.VMEM((48, N), jnp.float32),
        ],
    )(src, dst, el_g, er_g, m_g, h_g)


def _dense_h_kernel(in_ref, cov_ref, wd_ref, wm_ref, ws_ref,
                    ald_ref, alm_ref, als_ref, ard_ref, arm_ref, ars_ref,
                    hd_ref, hm_ref, hs_ref,
                    eld_ref, elm_ref, els_ref, erd_ref, erm_ref, ers_ref):
    x = in_ref[...] + cov_ref[...]
    for w_ref, al_ref, ar_ref, h_ref, el_ref, er_ref in (
            (wd_ref, ald_ref, ard_ref, hd_ref, eld_ref, erd_ref),
            (wm_ref, alm_ref, arm_ref, hm_ref, elm_ref, erm_ref),
            (ws_ref, als_ref, ars_ref, hs_ref, els_ref, ers_ref)):
        h = jnp.dot(x, w_ref[...], preferred_element_type=jnp.float32)
        el = jnp.dot(h, al_ref[...], preferred_element_type=jnp.float32)
        er = jnp.dot(h, ar_ref[...], preferred_element_type=jnp.float32)
        h_ref[...] = (h.reshape(BBLK, N, 48).transpose(0, 2, 1)
                      .reshape(BBLK * 48, N))
        el_ref[...] = (el.reshape(BBLK, N, 8).transpose(0, 2, 1)
                       .reshape(BBLK * 8, N))
        er_ref[...] = (er.reshape(BBLK, N, 8).transpose(0, 2, 1)
                       .reshape(BBLK * 8, N))


def _assemble_kernel(od_ref, om_ref, os_ref, resid_ref, bias_ref, out_ref):
    parts = []
    for head in range(H):
        for o_ref in (od_ref, om_ref, os_ref):
            parts.append(o_ref[pl.ds(head * HID, HID), :].transpose(1, 0))
    x = jnp.concatenate(parts, axis=1)
    out_ref[...] = resid_ref[...] + x + bias_ref[...]


def _attn_kernel(cov_ref, out_ref, acc_ref):
    t = pl.program_id(1)

    @pl.when(t == 0)
    def _():
        c = cov_ref[0]
        a = lax.dot_general(c, c, (((1,), (1,)), ((), ())),
                            preferred_element_type=jnp.float32)
        m = jnp.max(a, axis=1, keepdims=True)
        p = jnp.exp(a - m)
        acc_ref[...] = p / jnp.sum(p, axis=1, keepdims=True)

    out_ref[0] = acc_ref[...]


def _expand_al(al):
    flat = al.reshape(48)
    cols = jnp.arange(48) // HID
    onehot = (jnp.arange(8)[None, :] == cols[:, None]).astype(jnp.float32)
    return flat[:, None] * onehot


def kernel(input, covariate, edge_index_d, W_d, al_d, ar_d, b_d,
           edge_index_m, W_m, al_m, ar_m, b_m,
           edge_index_s, W_s, al_s, ar_s, b_s):
    batch, T = input.shape[0], input.shape[1]
    in_r = input.reshape(TB * N, F)
    cov_r = covariate.reshape(TB * N, F)

    blk = N * BBLK
    grid_a = (TB * N) // blk
    als = [_expand_al(a) for a in (al_d, al_m, al_s)]
    ars = [_expand_al(a) for a in (ar_d, ar_m, ar_s)]
    row_spec = pl.BlockSpec((blk, F), lambda i: (i, 0))
    w_spec = pl.BlockSpec((F, 48), lambda i: (0, 0))
    a_spec = pl.BlockSpec((48, 8), lambda i: (0, 0))
    h_spec = pl.BlockSpec((BBLK * 48, N), lambda i: (i, 0))
    e_spec = pl.BlockSpec((BBLK * 8, N), lambda i: (i, 0))
    res = pl.pallas_call(
        _dense_h_kernel,
        grid=(grid_a,),
        out_shape=[jax.ShapeDtypeStruct((TB * 48, N), jnp.float32)] * 3
        + [jax.ShapeDtypeStruct((TB * 8, N), jnp.float32)] * 6,
        in_specs=[row_spec, row_spec, w_spec, w_spec, w_spec,
                  a_spec, a_spec, a_spec, a_spec, a_spec, a_spec],
        out_specs=[h_spec] * 3 + [e_spec] * 6,
    )(in_r, cov_r, W_d, W_m, W_s, *als, *ars)
    h_gs = res[:3]
    el_gs = res[3:6]
    er_gs = res[6:9]

    out_scs = []
    for g, edge_index in enumerate([edge_index_d, edge_index_m, edge_index_s]):
        src = edge_index[0].astype(jnp.int32)
        dst = edge_index[1].astype(jnp.int32)
        el3 = el_gs[g].reshape(TB, 8, N)[:, :H]
        er3 = er_gs[g].reshape(TB, 8, N)[:, :H]
        ms = jnp.max(el3, axis=2) + jnp.max(er3, axis=2)
        m_g = jnp.broadcast_to(
            jnp.where(ms >= 0, ms, 0.2 * ms).reshape(NW, CPT, 1), (NW, CPT, 16))
        out_scs.append(_sc_gat_edge(src, dst, el_gs[g], er_gs[g], m_g, h_gs[g]))

    bias_cat = jnp.stack(
        [b.reshape(H, HID) for b in (b_d, b_m, b_s)], axis=1).reshape(1, F)

    def _q(bb, t):
        p = bb * T + t
        return (p % batch) * T + p // batch

    o_spec = pl.BlockSpec((48, N), lambda bb, t: (_q(bb, t), 0))
    out = pl.pallas_call(
        _assemble_kernel,
        grid=(batch, T),
        out_shape=jax.ShapeDtypeStruct((TB * N, F), jnp.float32),
        in_specs=[o_spec, o_spec, o_spec,
                  pl.BlockSpec((N, F), lambda bb, t: (bb * T + t, 0)),
                  pl.BlockSpec((1, F), lambda bb, t: (0, 0))],
        out_specs=pl.BlockSpec((N, F), lambda bb, t: (bb * T + t, 0)),
    )(*out_scs, in_r, bias_cat).reshape(batch, T, N, F)

    cov0 = covariate[:, 0]
    attn = pl.pallas_call(
        _attn_kernel,
        grid=(batch, T),
        out_shape=jax.ShapeDtypeStruct((batch * T, N, N), jnp.float32),
        in_specs=[pl.BlockSpec((1, N, F), lambda bb, t: (bb, 0, 0))],
        out_specs=pl.BlockSpec((1, N, N), lambda bb, t: (t * batch + bb, 0, 0)),
        scratch_shapes=[pltpu.VMEM((N, N), jnp.float32)],
    )(cov0)
    return out, attn
